# baseline (device time: 38560 ns/iter reference)
import jax
import jax.numpy as jnp
from jax import lax
from jax.experimental import pallas as pl
from jax.experimental.pallas import tpu as pltpu

_CompilerParams = getattr(pltpu, "CompilerParams", None) or getattr(
    pltpu, "TPUCompilerParams"
)


def kernel(x, Wdkv, Wuk, Wuv, Wq, Wqr, Wkr, Wo):
    B, S, D = x.shape
    BS = B * S
    dc = Wdkv.shape[1]
    H, Dh = 16, 64
    Dr = Wkr.shape[1]
    Dp = Dh + Dr
    scale = Dp ** -0.5
    bf16 = jnp.bfloat16
    f32 = jnp.float32

    def body(x_ref, wdkv_ref, wuk_ref, wuv_ref, wq_ref, wqr_ref, wkr_ref,
             wo_ref, out_ref, c_send, c_recv, w_send, w_recv, qq, kk, o_buf,
             send_sems, recv_sems):
        my_x = lax.axis_index("x")
        my_y = lax.axis_index("y")
        my_z = lax.axis_index("z")
        partner = (my_x, 1 - my_y, my_z)

        barrier = pltpu.get_barrier_semaphore()
        pl.semaphore_signal(barrier, inc=1, device_id=partner,
                            device_id_type=pl.DeviceIdType.MESH)
        pl.semaphore_wait(barrier, 1)

        w_send[:, 0:D] = wuk_ref[:].astype(bf16)
        w_send[:, D:2 * D] = wuv_ref[:].astype(bf16)
        rdma_w = pltpu.make_async_remote_copy(
            src_ref=w_send, dst_ref=w_recv,
            send_sem=send_sems.at[1], recv_sem=recv_sems.at[1],
            device_id=partner, device_id_type=pl.DeviceIdType.MESH,
        )
        rdma_w.start()

        xf = x_ref[:].reshape(BS, D).astype(bf16)
        c_send[:, :] = lax.dot(
            xf, wdkv_ref[:].astype(bf16), preferred_element_type=f32
        ).astype(bf16)
        rdma_c = pltpu.make_async_remote_copy(
            src_ref=c_send, dst_ref=c_recv,
            send_sem=send_sems.at[0], recv_sem=recv_sems.at[0],
            device_id=partner, device_id_type=pl.DeviceIdType.MESH,
        )
        rdma_c.start()

        Q = (lax.dot(xf, wq_ref[:].astype(bf16),
                     preferred_element_type=f32) * scale).astype(bf16)
        Qr = (lax.dot(xf, wqr_ref[:].astype(bf16),
                      preferred_element_type=f32) * scale).astype(bf16)
        Kr = lax.dot(xf, wkr_ref[:].astype(bf16),
                     preferred_element_type=f32).astype(bf16)

        for h in range(H):
            qq[:, h * Dp:h * Dp + Dh] = Q[:, h * Dh:(h + 1) * Dh]
            qq[:, h * Dp + Dh:(h + 1) * Dp] = Qr[:, h * Dr:(h + 1) * Dr]

        rdma_c.wait_recv()
        rdma_w.wait_recv()
        c_cat = jnp.concatenate([c_send[:, :], c_recv[:, :]], axis=1)
        w_cat = jnp.concatenate([w_send[:, :], w_recv[:, :]], axis=0)
        KV = lax.dot(c_cat, w_cat, preferred_element_type=f32)
        V = KV[:, D:2 * D].astype(bf16)

        for h in range(H):
            kk[:, h * Dp:h * Dp + Dh] = KV[:, h * Dh:(h + 1) * Dh].astype(bf16)
            kk[:, h * Dp + Dh:(h + 1) * Dp] = Kr

        for b in range(B):
            r0 = b * S
            for h in range(H):
                qh = qq[r0:r0 + S, h * Dp:(h + 1) * Dp]
                khh = kk[r0:r0 + S, h * Dp:(h + 1) * Dp]
                s = lax.dot_general(
                    qh, khh, (((1,), (1,)), ((), ())),
                    preferred_element_type=f32,
                )
                m = jnp.max(s, axis=-1, keepdims=True)
                p = jnp.exp(s - m)
                p = p * (1.0 / jnp.sum(p, axis=-1, keepdims=True))
                v = V[r0:r0 + S, h * Dh:(h + 1) * Dh]
                o = lax.dot(p.astype(bf16), v, preferred_element_type=f32)
                o_buf[r0:r0 + S, h * Dh:(h + 1) * Dh] = o.astype(bf16)

        out = lax.dot(o_buf[:, :], wo_ref[:].astype(bf16),
                      preferred_element_type=f32)
        out_ref[:] = out.reshape(B, S, D)

        rdma_c.wait_send()
        rdma_w.wait_send()

    vmem = pl.BlockSpec(memory_space=pltpu.VMEM)
    return pl.pallas_call(
        body,
        out_shape=jax.ShapeDtypeStruct((B, S, D), jnp.float32),
        in_specs=[vmem] * 8,
        out_specs=vmem,
        scratch_shapes=[
            pltpu.VMEM((BS, dc), bf16),
            pltpu.VMEM((BS, dc), bf16),
            pltpu.VMEM((dc, 2 * D), bf16),
            pltpu.VMEM((dc, 2 * D), bf16),
            pltpu.VMEM((BS, H * Dp), bf16),
            pltpu.VMEM((BS, H * Dp), bf16),
            pltpu.VMEM((BS, H * Dh), bf16),
            pltpu.SemaphoreType.DMA((2,)),
            pltpu.SemaphoreType.DMA((2,)),
        ],
        compiler_params=_CompilerParams(collective_id=0),
    )(x, Wdkv, Wuk, Wuv, Wq, Wqr, Wkr, Wo)


# device time: 30599 ns/iter; 1.2602x vs baseline; 1.2602x over previous
import jax
import jax.numpy as jnp
from jax import lax
from jax.experimental import pallas as pl
from jax.experimental.pallas import tpu as pltpu

_CompilerParams = getattr(pltpu, "CompilerParams", None) or getattr(
    pltpu, "TPUCompilerParams"
)


def kernel(x, Wdkv, Wuk, Wuv, Wq, Wqr, Wkr, Wo):
    B, S, D = x.shape
    BS = B * S
    dc = Wdkv.shape[1]
    H, Dh = 16, 64
    Dr = Wkr.shape[1]
    Dp = Dh + Dr
    Dv = 2 * Dh
    scale = Dp ** -0.5 * 1.4426950408889634
    bf16 = jnp.bfloat16
    f32 = jnp.float32

    def body(x_ref, wdkv_ref, wuk_ref, wuv_ref, wq_ref, wqr_ref, wkr_ref,
             wo_ref, out_ref, c_send, c_recv, w_send, w_recv, qq, kk, vv,
             o_buf, send_sems, recv_sems):
        my_x = lax.axis_index("x")
        my_y = lax.axis_index("y")
        my_z = lax.axis_index("z")
        partner = (my_x, 1 - my_y, my_z)

        barrier = pltpu.get_barrier_semaphore()
        pl.semaphore_signal(barrier, inc=1, device_id=partner,
                            device_id_type=pl.DeviceIdType.MESH)
        pl.semaphore_wait(barrier, 1)

        w_send[:, 0:D] = wuk_ref[:].astype(bf16)
        w_send[:, D:2 * D] = wuv_ref[:].astype(bf16)
        rdma_w = pltpu.make_async_remote_copy(
            src_ref=w_send, dst_ref=w_recv,
            send_sem=send_sems.at[1], recv_sem=recv_sems.at[1],
            device_id=partner, device_id_type=pl.DeviceIdType.MESH,
        )
        rdma_w.start()

        xf = x_ref[:].reshape(BS, D).astype(bf16)
        c_send[:, :] = lax.dot(
            xf, wdkv_ref[:].astype(bf16), preferred_element_type=f32
        ).astype(bf16)
        rdma_c = pltpu.make_async_remote_copy(
            src_ref=c_send, dst_ref=c_recv,
            send_sem=send_sems.at[0], recv_sem=recv_sems.at[0],
            device_id=partner, device_id_type=pl.DeviceIdType.MESH,
        )
        rdma_c.start()

        Q = (lax.dot(xf, wq_ref[:].astype(bf16),
                     preferred_element_type=f32) * scale).astype(bf16)
        Qr = (lax.dot(xf, wqr_ref[:].astype(bf16),
                      preferred_element_type=f32) * scale).astype(bf16)
        Kr = lax.dot(xf, wkr_ref[:].astype(bf16),
                     preferred_element_type=f32).astype(bf16)

        for h in range(H):
            qq[:, h * Dp:h * Dp + Dh] = Q[:, h * Dh:(h + 1) * Dh]
            qq[:, h * Dp + Dh:(h + 1) * Dp] = Qr[:, h * Dr:(h + 1) * Dr]

        rdma_c.wait_recv()
        rdma_w.wait_recv()
        c_cat = jnp.concatenate([c_send[:, :], c_recv[:, :]], axis=1)
        w_cat = jnp.concatenate([w_send[:, :], w_recv[:, :]], axis=0)
        KV = lax.dot(c_cat, w_cat, preferred_element_type=f32)

        ones_col = jnp.ones((BS, 1), bf16)
        for h in range(H):
            kk[:, h * Dp:h * Dp + Dh] = KV[:, h * Dh:(h + 1) * Dh].astype(bf16)
            kk[:, h * Dp + Dh:(h + 1) * Dp] = Kr
            vv[:, h * Dv:h * Dv + Dh] = KV[:, D + h * Dh:D + (h + 1) * Dh].astype(bf16)
            vv[:, h * Dv + Dh:h * Dv + Dh + 1] = ones_col

        for b in range(B):
            r0 = b * S
            for h in range(H):
                qh = qq[r0:r0 + S, h * Dp:(h + 1) * Dp]
                khh = kk[r0:r0 + S, h * Dp:(h + 1) * Dp]
                s = lax.dot_general(
                    qh, khh, (((1,), (1,)), ((), ())),
                    preferred_element_type=f32,
                )
                p = jnp.exp2(s).astype(bf16)
                v = vv[r0:r0 + S, h * Dv:h * Dv + Dh + 1]
                o = lax.dot(p, v, preferred_element_type=f32)
                r = 1.0 / o[:, Dh:Dh + 1]
                o_buf[r0:r0 + S, h * Dh:(h + 1) * Dh] = (o[:, 0:Dh] * r).astype(bf16)

        out = lax.dot(o_buf[:, :], wo_ref[:].astype(bf16),
                      preferred_element_type=f32)
        out_ref[:] = out.reshape(B, S, D)

        rdma_c.wait_send()
        rdma_w.wait_send()

    vmem = pl.BlockSpec(memory_space=pltpu.VMEM)
    return pl.pallas_call(
        body,
        out_shape=jax.ShapeDtypeStruct((B, S, D), jnp.float32),
        in_specs=[vmem] * 8,
        out_specs=vmem,
        scratch_shapes=[
            pltpu.VMEM((BS, dc), bf16),
            pltpu.VMEM((BS, dc), bf16),
            pltpu.VMEM((dc, 2 * D), bf16),
            pltpu.VMEM((dc, 2 * D), bf16),
            pltpu.VMEM((BS, H * Dp), bf16),
            pltpu.VMEM((BS, H * Dp), bf16),
            pltpu.VMEM((BS, H * Dv), bf16),
            pltpu.VMEM((BS, H * Dh), bf16),
            pltpu.SemaphoreType.DMA((2,)),
            pltpu.SemaphoreType.DMA((2,)),
        ],
        compiler_params=_CompilerParams(collective_id=0),
    )(x, Wdkv, Wuk, Wuv, Wq, Wqr, Wkr, Wo)
